# baseline (speedup 1.0000x reference)
import functools

import jax
import jax.numpy as jnp
from jax import lax
from jax.experimental import pallas as pl
from jax.experimental.pallas import tpu as pltpu
from jax.experimental.pallas import tpu_sc as plsc

N = 10000
D = 128
DG = 64
E = 320000

NC = 2
NS = 16
NT = NC * NS
NP = 10240
RT = NP // NS
K = 128
TCH = E // K
CB = TCH // NT
REM = TCH - CB * NT
C = CB + 2
PAD_ROWS = NP - N
NBUF = 8
LEAD = 4

f32 = jnp.float32
bf16 = jnp.bfloat16

_mesh = plsc.VectorSubcoreMesh(
    core_axis_name="c", subcore_axis_name="s", num_cores=NC, num_subcores=NS)


def _stage_slab(ei_hbm, which, slab_v, wid):
    base = wid * CB + jnp.minimum(wid, REM)
    pltpu.sync_copy(ei_hbm.at[which, pl.ds(base, CB)], slab_v.at[pl.ds(0, CB)])

    @pl.when(wid < REM)
    def _():
        pltpu.sync_copy(ei_hbm.at[which, pl.ds(base + CB, 1)],
                        slab_v.at[pl.ds(CB, 1)])

    def fill_row(r):
        for j in range(K // 16):
            vec = N + ((wid * K + r * 64 + j * 16
                        + lax.iota(jnp.int32, 16)) % PAD_ROWS)
            slab_v[r, pl.ds(j * 16, 16)] = vec

    @pl.when(wid >= REM)
    def _():
        fill_row(CB)

    fill_row(CB + 1)


@functools.partial(
    pl.kernel,
    out_type=jax.ShapeDtypeStruct((NC, NP), f32),
    mesh=_mesh,
    scratch_types=[
        pltpu.VMEM((C, K), jnp.int32),
        pltpu.VMEM((K,), f32),
        pltpu.VMEM((RT,), f32),
        pltpu.VMEM_SHARED((NP,), f32),
        pltpu.SemaphoreType.DMA,
    ],
    compiler_params=pltpu.CompilerParams(use_tc_tiling_on_sc=False),
)
def _deg_call(ei_hbm, ones_hbm, zrow_hbm, out_hbm, dst_v, ones_v, z_v,
              deg_sh, ssem):
    c = lax.axis_index("c")
    s = lax.axis_index("s")
    wid = s * NC + c
    rs = s * RT
    _stage_slab(ei_hbm, 1, dst_v, wid)
    pltpu.sync_copy(ones_hbm, ones_v)
    pltpu.sync_copy(zrow_hbm, z_v)
    pltpu.sync_copy(z_v, deg_sh.at[pl.ds(rs, RT)])
    plsc.subcore_barrier()

    Q = 8
    for j in range(Q):
        pltpu.async_copy(ones_v, deg_sh.at[dst_v.at[j]], ssem, add=True)

    def body(ch, carry):
        pltpu.make_async_copy(ones_v, deg_sh.at[dst_v.at[ch]], ssem).wait()
        pltpu.async_copy(ones_v, deg_sh.at[dst_v.at[ch + Q]], ssem, add=True)
        return carry

    lax.fori_loop(0, C - Q, body, 0)
    for j in range(C - Q, C):
        pltpu.make_async_copy(ones_v, deg_sh.at[dst_v.at[j]], ssem).wait()
    plsc.subcore_barrier()
    pltpu.sync_copy(deg_sh.at[pl.ds(rs, RT)], out_hbm.at[c, pl.ds(rs, RT)])


@functools.partial(
    pl.kernel,
    out_type=jax.ShapeDtypeStruct((NC, NP, DG), bf16),
    mesh=_mesh,
    scratch_types=[
        pltpu.VMEM((C, K), jnp.int32),
        pltpu.VMEM((C, K), jnp.int32),
        *([pltpu.VMEM((K, DG), bf16)] * NBUF),
        pltpu.VMEM_SHARED((NP, DG), bf16),
        *([pltpu.SemaphoreType.DMA] * (2 * NBUF)),
    ],
    compiler_params=pltpu.CompilerParams(use_tc_tiling_on_sc=False),
)
def _scat_call(g_hbm, ei_hbm, zrows_hbm, out_hbm, *refs):
    src_v, dst_v = refs[0], refs[1]
    rows = refs[2:2 + NBUF]
    acc_sh = refs[2 + NBUF]
    gsem = refs[3 + NBUF:3 + 2 * NBUF]
    ssem = refs[3 + 2 * NBUF:3 + 3 * NBUF]
    c = lax.axis_index("c")
    s = lax.axis_index("s")
    wid = s * NC + c
    rs = s * RT
    _stage_slab(ei_hbm, 0, src_v, wid)
    _stage_slab(ei_hbm, 1, dst_v, wid)

    @pl.when(c == 0)
    def _():
        pltpu.sync_copy(g_hbm.at[pl.ds(rs, RT)], acc_sh.at[pl.ds(rs, RT)])

    @pl.when(c != 0)
    def _():
        pltpu.sync_copy(zrows_hbm, acc_sh.at[pl.ds(rs, RT)])

    plsc.subcore_barrier()

    for j in range(LEAD):
        pltpu.async_copy(g_hbm.at[src_v.at[j]], rows[j], gsem[j])

    SLACK = NBUF - LEAD

    def body(i, carry):
        for b in range(NBUF):
            ch = NBUF * i + b
            bb = (b + LEAD) % NBUF

            @pl.when(ch + LEAD < C)
            def _():
                @pl.when(ch >= SLACK)
                def _():
                    pltpu.make_async_copy(
                        rows[bb], acc_sh.at[dst_v.at[ch - SLACK]],
                        ssem[bb]).wait()

                pltpu.async_copy(g_hbm.at[src_v.at[ch + LEAD]],
                                 rows[bb], gsem[bb])

            pltpu.make_async_copy(g_hbm.at[src_v.at[ch]], rows[b], gsem[b]).wait()
            pltpu.async_copy(rows[b], acc_sh.at[dst_v.at[ch]], ssem[b], add=True)

        return carry

    lax.fori_loop(0, C // NBUF, body, 0)
    for ch in range(C - NBUF, C):
        b = ch % NBUF
        pltpu.make_async_copy(rows[b], acc_sh.at[dst_v.at[ch]], ssem[b]).wait()
    plsc.subcore_barrier()
    pltpu.sync_copy(acc_sh.at[pl.ds(rs, RT)], out_hbm.at[c, pl.ds(rs, RT)])


NH = NP // 2
NHR = N // 2


def _dinv_expand(dinv2):
    return jnp.concatenate(
        [jnp.broadcast_to(dinv2[:, 0:1], (NH, DG)),
         jnp.broadcast_to(dinv2[:, 1:2], (NH, DG))], axis=1)


def _tc1a_body(xP_ref, W0_ref, b0_ref, h0_ref):
    W0 = W0_ref[...]
    b0 = b0_ref[...]
    h_even = jnp.dot(xP_ref[:, DG:D], W0, preferred_element_type=f32) + b0
    h_odd = jnp.dot(xP_ref[:, D + DG:], W0, preferred_element_type=f32) + b0
    h0_ref[:NHR] = jnp.concatenate([h_even, h_odd], axis=1)
    h0_ref[NHR:] = jnp.broadcast_to(jnp.concatenate([b0, b0], axis=1),
                                    (NH - NHR, D))


def _tc1b_body(degp_ref, h0_ref, g0_ref, dinv2_ref):
    degsum = degp_ref[0] + degp_ref[1] + 1.0
    dinv2 = lax.rsqrt(degsum)
    g0_ref[...] = (h0_ref[...] * _dinv_expand(dinv2)).astype(bf16)
    dinv2_ref[...] = dinv2


def _tc2_body(xP_ref, acc_ref, dinv2_ref, W1_ref, b1_ref, y0_ref, g1_ref):
    accs = acc_ref[0].astype(f32) + acc_ref[1].astype(f32)
    dexp = _dinv_expand(dinv2_ref[...])
    fm = jnp.maximum(accs * dexp, 0.0)
    y0 = jnp.concatenate([xP_ref[:, :DG] + fm[:NHR, :DG],
                          xP_ref[:, D:D + DG] + fm[:NHR, DG:]], axis=1)
    y0_ref[...] = y0
    W1 = W1_ref[...]
    b1 = b1_ref[...]
    h1_even = jnp.dot(y0[:, :DG], W1, preferred_element_type=f32) + b1
    h1_odd = jnp.dot(y0[:, DG:], W1, preferred_element_type=f32) + b1
    g1_ref[:NHR] = (jnp.concatenate([h1_even, h1_odd], axis=1)
                    * dexp[:NHR]).astype(bf16)
    g1_ref[NHR:] = (jnp.broadcast_to(jnp.concatenate([b1, b1], axis=1),
                                     (NH - NHR, D)) * dexp[NHR:]).astype(bf16)


def _tc3_body(xP_ref, acc_ref, dinv2_ref, y0_ref, out_ref):
    accs = acc_ref[0].astype(f32) + acc_ref[1].astype(f32)
    dexp = _dinv_expand(dinv2_ref[...])
    fm = jnp.maximum(accs * dexp, 0.0)
    y0 = y0_ref[...]
    out_ref[:, 0:DG] = y0[:, :DG]
    out_ref[:, DG:D] = xP_ref[:, DG:D] + fm[:NHR, :DG]
    out_ref[:, D:D + DG] = y0[:, DG:]
    out_ref[:, D + DG:] = xP_ref[:, D + DG:] + fm[:NHR, DG:]


_tc1a = pl.pallas_call(
    _tc1a_body,
    out_shape=jax.ShapeDtypeStruct((NH, D), f32),
)

_tc1b = pl.pallas_call(
    _tc1b_body,
    out_shape=[jax.ShapeDtypeStruct((NH, D), bf16),
               jax.ShapeDtypeStruct((NH, 2), f32)],
)

_tc2 = pl.pallas_call(
    _tc2_body,
    out_shape=[jax.ShapeDtypeStruct((NHR, D), f32),
               jax.ShapeDtypeStruct((NH, D), bf16)],
)

_tc3 = pl.pallas_call(
    _tc3_body,
    out_shape=jax.ShapeDtypeStruct((NHR, 2 * D), f32),
)


def kernel(x, edge_index, W0, b0, W1, b1):
    x = x.astype(f32)
    xP = x.reshape(NHR, 2 * D)
    ei3 = edge_index.astype(jnp.int32).reshape(2, TCH, K)

    ones_k = jnp.ones((K,), f32)
    zrow = jnp.zeros((RT,), f32)
    zrows = jnp.zeros((RT, DG), bf16)

    degp = _deg_call(ei3, ones_k, zrow)
    degp2 = degp.reshape(NC, NH, 2)

    h0 = _tc1a(xP, W0, b0.reshape(1, DG))
    g0, dinv2 = _tc1b(degp2, h0)
    acc0 = _scat_call(g0.reshape(NP, DG), ei3, zrows)
    y0, g1 = _tc2(xP, acc0.reshape(NC, NH, D), dinv2, W1, b1.reshape(1, DG))
    acc1 = _scat_call(g1.reshape(NP, DG), ei3, zrows)
    outP = _tc3(xP, acc1.reshape(NC, NH, D), dinv2, y0)
    return outP.reshape(N, D)


# speedup vs baseline: 1.0159x; 1.0159x over previous
import functools

import jax
import jax.numpy as jnp
from jax import lax
from jax.experimental import pallas as pl
from jax.experimental.pallas import tpu as pltpu
from jax.experimental.pallas import tpu_sc as plsc

N = 10000
D = 128
DG = 64
E = 320000

NC = 2
NS = 16
NT = NC * NS
NP = 10240
RT = NP // NS
K = 128
TCH = E // K
CB = TCH // NT
REM = TCH - CB * NT
C = CB + 2
PAD_ROWS = NP - N
NBUF = 10
LEAD = 5

f32 = jnp.float32
bf16 = jnp.bfloat16

_mesh = plsc.VectorSubcoreMesh(
    core_axis_name="c", subcore_axis_name="s", num_cores=NC, num_subcores=NS)


def _stage_slab(ei_hbm, which, slab_v, wid):
    base = wid * CB + jnp.minimum(wid, REM)
    pltpu.sync_copy(ei_hbm.at[which, pl.ds(base, CB)], slab_v.at[pl.ds(0, CB)])

    @pl.when(wid < REM)
    def _():
        pltpu.sync_copy(ei_hbm.at[which, pl.ds(base + CB, 1)],
                        slab_v.at[pl.ds(CB, 1)])

    def fill_row(r):
        for j in range(K // 16):
            vec = N + ((wid * K + r * 64 + j * 16
                        + lax.iota(jnp.int32, 16)) % PAD_ROWS)
            slab_v[r, pl.ds(j * 16, 16)] = vec

    @pl.when(wid >= REM)
    def _():
        fill_row(CB)

    fill_row(CB + 1)


@functools.partial(
    pl.kernel,
    out_type=jax.ShapeDtypeStruct((NC, NP), f32),
    mesh=_mesh,
    scratch_types=[
        pltpu.VMEM((C, K), jnp.int32),
        pltpu.VMEM((K,), f32),
        pltpu.VMEM((RT,), f32),
        pltpu.VMEM_SHARED((NP,), f32),
        pltpu.SemaphoreType.DMA,
    ],
    compiler_params=pltpu.CompilerParams(use_tc_tiling_on_sc=False),
)
def _deg_call(ei_hbm, ones_hbm, zrow_hbm, out_hbm, dst_v, ones_v, z_v,
              deg_sh, ssem):
    c = lax.axis_index("c")
    s = lax.axis_index("s")
    wid = s * NC + c
    rs = s * RT
    _stage_slab(ei_hbm, 1, dst_v, wid)
    pltpu.sync_copy(ones_hbm, ones_v)
    pltpu.sync_copy(zrow_hbm, z_v)
    pltpu.sync_copy(z_v, deg_sh.at[pl.ds(rs, RT)])
    plsc.subcore_barrier()

    Q = 8
    for j in range(Q):
        pltpu.async_copy(ones_v, deg_sh.at[dst_v.at[j]], ssem, add=True)

    def body(ch, carry):
        pltpu.make_async_copy(ones_v, deg_sh.at[dst_v.at[ch]], ssem).wait()
        pltpu.async_copy(ones_v, deg_sh.at[dst_v.at[ch + Q]], ssem, add=True)
        return carry

    lax.fori_loop(0, C - Q, body, 0)
    for j in range(C - Q, C):
        pltpu.make_async_copy(ones_v, deg_sh.at[dst_v.at[j]], ssem).wait()
    plsc.subcore_barrier()
    pltpu.sync_copy(deg_sh.at[pl.ds(rs, RT)], out_hbm.at[c, pl.ds(rs, RT)])


@functools.partial(
    pl.kernel,
    out_type=jax.ShapeDtypeStruct((NC, NP, DG), bf16),
    mesh=_mesh,
    scratch_types=[
        pltpu.VMEM((C, K), jnp.int32),
        pltpu.VMEM((C, K), jnp.int32),
        *([pltpu.VMEM((K, DG), bf16)] * NBUF),
        pltpu.VMEM_SHARED((NP, DG), bf16),
        *([pltpu.SemaphoreType.DMA] * (2 * NBUF)),
    ],
    compiler_params=pltpu.CompilerParams(use_tc_tiling_on_sc=False),
)
def _scat_call(g_hbm, ei_hbm, zrows_hbm, out_hbm, *refs):
    src_v, dst_v = refs[0], refs[1]
    rows = refs[2:2 + NBUF]
    acc_sh = refs[2 + NBUF]
    gsem = refs[3 + NBUF:3 + 2 * NBUF]
    ssem = refs[3 + 2 * NBUF:3 + 3 * NBUF]
    c = lax.axis_index("c")
    s = lax.axis_index("s")
    wid = s * NC + c
    rs = s * RT
    _stage_slab(ei_hbm, 0, src_v, wid)
    _stage_slab(ei_hbm, 1, dst_v, wid)

    @pl.when(c == 0)
    def _():
        pltpu.sync_copy(g_hbm.at[pl.ds(rs, RT)], acc_sh.at[pl.ds(rs, RT)])

    @pl.when(c != 0)
    def _():
        pltpu.sync_copy(zrows_hbm, acc_sh.at[pl.ds(rs, RT)])

    plsc.subcore_barrier()

    for j in range(LEAD):
        pltpu.async_copy(g_hbm.at[src_v.at[j]], rows[j], gsem[j])

    SLACK = NBUF - LEAD

    def body(i, carry):
        for b in range(NBUF):
            ch = NBUF * i + b
            bb = (b + LEAD) % NBUF

            @pl.when(ch + LEAD < C)
            def _():
                @pl.when(ch >= SLACK)
                def _():
                    pltpu.make_async_copy(
                        rows[bb], acc_sh.at[dst_v.at[ch - SLACK]],
                        ssem[bb]).wait()

                pltpu.async_copy(g_hbm.at[src_v.at[ch + LEAD]],
                                 rows[bb], gsem[bb])

            pltpu.make_async_copy(g_hbm.at[src_v.at[ch]], rows[b], gsem[b]).wait()
            pltpu.async_copy(rows[b], acc_sh.at[dst_v.at[ch]], ssem[b], add=True)

        return carry

    lax.fori_loop(0, C // NBUF, body, 0)
    for ch in range(C - NBUF, C):
        b = ch % NBUF
        pltpu.make_async_copy(rows[b], acc_sh.at[dst_v.at[ch]], ssem[b]).wait()
    plsc.subcore_barrier()
    pltpu.sync_copy(acc_sh.at[pl.ds(rs, RT)], out_hbm.at[c, pl.ds(rs, RT)])


def _tc1a_body(x_ref, W0_ref, b0_ref, h0_ref):
    h0_ref[:N] = jnp.dot(x_ref[:, DG:], W0_ref[...],
                         preferred_element_type=f32) + b0_ref[...]
    h0_ref[N:] = jnp.broadcast_to(b0_ref[...], (NP - N, DG))


def _tc1b_body(degT_ref, h0_ref, g0_ref, dinv_ref):
    deg = degT_ref[:, 0:1] + degT_ref[:, 1:2] + 1.0
    dinv = lax.rsqrt(deg)
    g0_ref[...] = (h0_ref[...] * dinv).astype(bf16)
    dinv_ref[...] = dinv


def _tc2_body(x_ref, acc_ref, dinv_ref, W1_ref, b1_ref, y0_ref, g1_ref):
    accs = acc_ref[0, :N].astype(f32) + acc_ref[1, :N].astype(f32)
    fm = jnp.maximum(accs * dinv_ref[:N], 0.0)
    y0 = x_ref[:, :DG] + fm
    y0_ref[...] = y0
    h1 = jnp.dot(y0, W1_ref[...], preferred_element_type=f32) + b1_ref[...]
    g1_ref[:N] = (h1 * dinv_ref[:N]).astype(bf16)
    g1_ref[N:] = (jnp.broadcast_to(b1_ref[...], (NP - N, DG))
                  * dinv_ref[N:]).astype(bf16)


def _tc3_body(x_ref, acc_ref, dinv_ref, y0_ref, out_ref):
    accs = acc_ref[0, :N].astype(f32) + acc_ref[1, :N].astype(f32)
    fm = jnp.maximum(accs * dinv_ref[:N], 0.0)
    out_ref[:, :DG] = y0_ref[...]
    out_ref[:, DG:] = x_ref[:, DG:] + fm


_tc1a = pl.pallas_call(
    _tc1a_body,
    out_shape=jax.ShapeDtypeStruct((NP, DG), f32),
)

_tc1b = pl.pallas_call(
    _tc1b_body,
    out_shape=[jax.ShapeDtypeStruct((NP, DG), bf16),
               jax.ShapeDtypeStruct((NP, 1), f32)],
)

_tc2 = pl.pallas_call(
    _tc2_body,
    out_shape=[jax.ShapeDtypeStruct((N, DG), f32),
               jax.ShapeDtypeStruct((NP, DG), bf16)],
)

_tc3 = pl.pallas_call(
    _tc3_body,
    out_shape=jax.ShapeDtypeStruct((N, D), f32),
)


def kernel(x, edge_index, W0, b0, W1, b1):
    x = x.astype(f32)
    ei3 = edge_index.astype(jnp.int32).reshape(2, TCH, K)

    ones_k = jnp.ones((K,), f32)
    zrow = jnp.zeros((RT,), f32)
    zrows = jnp.zeros((RT, DG), bf16)

    degp = _deg_call(ei3, ones_k, zrow)
    degT = degp.T

    h0 = _tc1a(x, W0, b0.reshape(1, DG))
    g0, dinv = _tc1b(degT, h0)
    acc0 = _scat_call(g0, ei3, zrows)
    y0, g1 = _tc2(x, acc0, dinv, W1, b1.reshape(1, DG))
    acc1 = _scat_call(g1, ei3, zrows)
    return _tc3(x, acc1, dinv, y0)
